# baseline (speedup 1.0000x reference)
import functools

import jax
import jax.numpy as jnp
from jax import lax
from jax.experimental import pallas as pl
from jax.experimental.pallas import tpu as pltpu
from jax.experimental.pallas import tpu_sc as plsc

N = 10000
D = 128
E = 320000
H = 256
ACT = 8

NC = 2
NS = 16
NW = NC * NS
CH = 80
EP = 10240
EPAD = NW * EP
NCH = EP // CH
NPAD = 10240
RPT = NPAD // NS
DUMMY = N + 100

def _deg_body(dst_hbm, out_hbm, dstbuf, locdeg):
    c = lax.axis_index("c")
    s = lax.axis_index("s")
    wid = s * NC + c
    zero16 = jnp.zeros((16,), jnp.float32)
    ones16 = jnp.ones((16,), jnp.float32)

    def zb(i, carry):
        locdeg[pl.ds(i * 16, 16)] = zero16
        return carry

    lax.fori_loop(0, NPAD // 16, zb, 0)
    pltpu.sync_copy(dst_hbm.at[pl.ds(wid * EP, EP)], dstbuf)

    def ab(i, carry):
        idx = dstbuf[pl.ds(i * 16, 16)]
        plsc.addupdate_scatter(locdeg, [idx], ones16)
        return carry

    lax.fori_loop(0, EP // 16, ab, 0)
    pltpu.sync_copy(locdeg, out_hbm.at[c].at[s])


NBUF = 4


def _msg_body(src_hbm, dst_hbm, z_hbm, out_hbm, sidx, didx, rows, acc,
              gs0, gs1, gs2, gs3, ss0, ss1, ss2, ss3):
    c = lax.axis_index("c")
    s = lax.axis_index("s")
    wid = s * NC + c
    base = wid * EP
    zero16 = jnp.zeros((16,), jnp.float32)
    gsem = (gs0, gs1, gs2, gs3)
    ssem = (ss0, ss1, ss2, ss3)

    def zb(i, carry):
        r = i // (D // 16)
        k = i % (D // 16)
        rows[0, r, pl.ds(k * 16, 16)] = zero16
        return carry

    lax.fori_loop(0, CH * D // 16, zb, 0)
    for q in range(RPT // CH):
        pltpu.sync_copy(rows.at[0], acc.at[pl.ds(s * RPT + q * CH, CH)])
    plsc.subcore_barrier()

    def load_and_gather(j, b):
        pltpu.sync_copy(src_hbm.at[pl.ds(base + j * CH, CH)], sidx.at[b])
        pltpu.sync_copy(dst_hbm.at[pl.ds(base + j * CH, CH)], didx.at[b])
        pltpu.make_async_copy(z_hbm.at[sidx.at[b]], rows.at[b],
                              gsem[b]).start()

    for b in range(NBUF - 1):
        load_and_gather(b, b)

    def mb(g, carry):
        for b in range(NBUF):
            j = g * NBUF + b
            b3 = (b + NBUF - 1) % NBUF
            @pl.when((j + NBUF - 1 < NCH) & (j >= 1))
            def _drain():
                pltpu.make_async_copy(rows.at[b3], acc.at[didx.at[b3]],
                                      ssem[b3]).wait()

            @pl.when(j + NBUF - 1 < NCH)
            def _prefetch():
                load_and_gather(j + NBUF - 1, b3)

            pltpu.make_async_copy(z_hbm.at[sidx.at[b]], rows.at[b],
                                  gsem[b]).wait()
            pltpu.async_copy(rows.at[b], acc.at[didx.at[b]], ssem[b],
                             add=True)
        return carry

    lax.fori_loop(0, NCH // NBUF, mb, 0)
    for b in range(NBUF):
        pltpu.make_async_copy(rows.at[b], acc.at[didx.at[b]], ssem[b]).wait()
    plsc.subcore_barrier()
    pltpu.sync_copy(acc.at[pl.ds(s * RPT, RPT)],
                    out_hbm.at[c].at[pl.ds(s * RPT, RPT)])


@functools.cache
def _sc_kernels():
    mesh = plsc.VectorSubcoreMesh(core_axis_name="c", subcore_axis_name="s")
    cparams = pltpu.CompilerParams(needs_layout_passes=False)
    deg_kernel = pl.kernel(
        _deg_body,
        out_type=jax.ShapeDtypeStruct((NC, NS, NPAD), jnp.float32),
        mesh=mesh,
        scratch_types=[
            pltpu.VMEM((EP,), jnp.int32),
            pltpu.VMEM((NPAD,), jnp.float32),
        ],
        compiler_params=cparams,
    )
    msg_kernel = pl.kernel(
        _msg_body,
        out_type=jax.ShapeDtypeStruct((NC, NPAD, D), jnp.float32),
        mesh=mesh,
        scratch_types=(
            [pltpu.VMEM((NBUF, CH), jnp.int32),
             pltpu.VMEM((NBUF, CH), jnp.int32),
             pltpu.VMEM((NBUF, CH, D), jnp.float32),
             pltpu.VMEM_SHARED((NPAD, D), jnp.float32)]
            + [pltpu.SemaphoreType.DMA] * (2 * NBUF)),
        compiler_params=cparams,
    )
    return deg_kernel, msg_kernel


def _dinv_body(dp_ref, dinv_ref):
    tot = jnp.sum(dp_ref[...], axis=0, keepdims=True) + 1.0
    dinv_ref[...] = lax.rsqrt(tot)


def _dinv_call(dp):
    return pl.pallas_call(
        _dinv_body,
        out_shape=jax.ShapeDtypeStruct((1, NPAD), jnp.float32),
    )(dp)


_BM = 2000


def _xw_body(state_ref, w_ref, xw_ref):
    xw_ref[...] = jnp.dot(state_ref[...], w_ref[...],
                          preferred_element_type=jnp.float32,
                          precision=lax.Precision.HIGHEST)


def _xw_call(state, w_gcn):
    grid = (N // _BM,)
    return pl.pallas_call(
        _xw_body,
        grid=grid,
        in_specs=[
            pl.BlockSpec((_BM, D), lambda i: (i, 0)),
            pl.BlockSpec((D, D), lambda i: (0, 0)),
        ],
        out_specs=pl.BlockSpec((_BM, D), lambda i: (i, 0)),
        out_shape=jax.ShapeDtypeStruct((N, D), jnp.float32),
    )(state, w_gcn)


def _zscale_body(xw_ref, dinv_ref, z_ref):
    z_ref[...] = xw_ref[...] * dinv_ref[...]


def _zscale_call(xw, dinv_col):
    grid = (N // _BM,)
    return pl.pallas_call(
        _zscale_body,
        grid=grid,
        in_specs=[
            pl.BlockSpec((_BM, D), lambda i: (i, 0)),
            pl.BlockSpec((_BM, 1), lambda i: (i, 0)),
        ],
        out_specs=pl.BlockSpec((_BM, D), lambda i: (i, 0)),
        out_shape=jax.ShapeDtypeStruct((N, D), jnp.float32),
    )(xw, dinv_col)


_BMH = 2048
_GH = _BMH // ACT


def _head_body(acc0, acc1, z, state, dinv, aflat, bg, w1, b1, w2, b2, w3, b3,
               out):
    x = (acc0[...] + acc1[...] + z[...]) * dinv[...] + bg[...]
    x = jnp.maximum(x, 0.0) + state[...]
    gi = lax.broadcasted_iota(jnp.int32, (_GH, _BMH), 0)
    ri = lax.broadcasted_iota(jnp.int32, (_GH, _BMH), 1)
    sel = (ri // ACT) == gi
    smat = jnp.where(sel, aflat[...] * 10.0, 0.0)
    y = jnp.dot(smat, x, preferred_element_type=jnp.float32, precision=lax.Precision.HIGHEST)
    h = jnp.maximum(
        jnp.dot(y, w1[...], preferred_element_type=jnp.float32, precision=lax.Precision.HIGHEST) + b1[...], 0.0)
    h = jnp.maximum(
        jnp.dot(h, w2[...], preferred_element_type=jnp.float32, precision=lax.Precision.HIGHEST) + b2[...], 0.0)
    out[...] = jnp.dot(h, w3[...], preferred_element_type=jnp.float32, precision=lax.Precision.HIGHEST) + b3[...]


def _head_call(acc0, acc1, z, state, dinv_col, aflat, b_gcn, w1, b1, w2, b2,
               w3, b3):
    grid = (NPAD // _BMH,)
    row_spec = pl.BlockSpec((_BMH, D), lambda i: (i, 0))
    return pl.pallas_call(
        _head_body,
        grid=grid,
        in_specs=[
            row_spec, row_spec, row_spec, row_spec,
            pl.BlockSpec((_BMH, 1), lambda i: (i, 0)),
            pl.BlockSpec((1, _BMH), lambda i: (0, i)),
            pl.BlockSpec((1, D), lambda i: (0, 0)),
            pl.BlockSpec((D, H), lambda i: (0, 0)),
            pl.BlockSpec((1, H), lambda i: (0, 0)),
            pl.BlockSpec((H, H), lambda i: (0, 0)),
            pl.BlockSpec((1, H), lambda i: (0, 0)),
            pl.BlockSpec((H, 1), lambda i: (0, 0)),
            pl.BlockSpec((1, 1), lambda i: (0, 0)),
        ],
        out_specs=pl.BlockSpec((_GH, 1), lambda i: (i, 0)),
        out_shape=jax.ShapeDtypeStruct((NPAD // ACT, 1), jnp.float32),
    )(acc0, acc1, z, state, dinv_col, aflat, b_gcn, w1, b1, w2, b2, w3, b3)


def kernel(state, edge_index, action, W_gcn, b_gcn, W1, b1, W2, b2, W3, b3):
    src = edge_index[0]
    dst = edge_index[1]
    wpad = EP - E // NW
    src_p = jnp.concatenate(
        [src.reshape(NW, E // NW),
         jnp.zeros((NW, wpad), jnp.int32)], axis=1).reshape(-1)
    dummy_rows = jnp.broadcast_to(
        jnp.arange(N, N + wpad, dtype=jnp.int32), (NW, wpad))
    dst_p = jnp.concatenate(
        [dst.reshape(NW, E // NW), dummy_rows], axis=1).reshape(-1)

    deg_kernel, msg_kernel = _sc_kernels()
    deg_parts = deg_kernel(dst_p)
    xw = _xw_call(state, W_gcn)
    dinv_row = _dinv_call(deg_parts.reshape(NW, NPAD))
    dinv_full = dinv_row.reshape(NPAD, 1)
    dinv_col = dinv_full[:N]

    z = _zscale_call(xw, dinv_col)
    accp = msg_kernel(src_p, dst_p, z)

    rpad = NPAD - N
    zrows = jnp.zeros((rpad, D), jnp.float32)
    z_p = jnp.concatenate([z, zrows])
    state_p = jnp.concatenate([state, zrows])
    aflat = jnp.concatenate(
        [action.reshape(1, N), jnp.zeros((1, rpad), jnp.float32)], axis=1)

    out = _head_call(
        accp[0], accp[1], z_p, state_p, dinv_full, aflat,
        b_gcn.reshape(1, D),
        W1, b1.reshape(1, H), W2, b2.reshape(1, H), W3, b3.reshape(1, 1))
    return out.reshape(NPAD // ACT)[:N // ACT]


# speedup vs baseline: 1.0693x; 1.0693x over previous
import functools

import jax
import jax.numpy as jnp
from jax import lax
from jax.experimental import pallas as pl
from jax.experimental.pallas import tpu as pltpu
from jax.experimental.pallas import tpu_sc as plsc

N = 10000
D = 128
E = 320000
H = 256
ACT = 8

NC = 2
NS = 16
NW = NC * NS
CH = 128
EP = 10240
EPAD = NW * EP
NCH = EP // CH
NPAD = 10240
RPT = NPAD // NS
DUMMY = N + 100

def _deg_body(dst_hbm, out_hbm, dstbuf, locdeg):
    c = lax.axis_index("c")
    s = lax.axis_index("s")
    wid = s * NC + c
    zero16 = jnp.zeros((16,), jnp.float32)
    ones16 = jnp.ones((16,), jnp.float32)

    def zb(i, carry):
        locdeg[pl.ds(i * 16, 16)] = zero16
        return carry

    lax.fori_loop(0, NPAD // 16, zb, 0)
    pltpu.sync_copy(dst_hbm.at[pl.ds(wid * EP, EP)], dstbuf)

    def ab(i, carry):
        idx = dstbuf[pl.ds(i * 16, 16)]
        plsc.addupdate_scatter(locdeg, [idx], ones16)
        return carry

    lax.fori_loop(0, EP // 16, ab, 0)
    pltpu.sync_copy(locdeg, out_hbm.at[c].at[s])


NBUF = 2


def _msg_body(src_hbm, dst_hbm, z_hbm, out_hbm, sidx, didx, rows, acc,
              gs0, gs1, ss0, ss1):
    c = lax.axis_index("c")
    s = lax.axis_index("s")
    wid = s * NC + c
    base = wid * EP
    zero16 = jnp.zeros((16,), jnp.float32)
    gsem = (gs0, gs1)
    ssem = (ss0, ss1)

    def zb(i, carry):
        r = i // (D // 16)
        k = i % (D // 16)
        rows[0, r, pl.ds(k * 16, 16)] = zero16
        return carry

    lax.fori_loop(0, CH * D // 16, zb, 0)
    for q in range(RPT // CH):
        pltpu.sync_copy(rows.at[0], acc.at[pl.ds(s * RPT + q * CH, CH)])
    plsc.subcore_barrier()

    def load_and_gather(j, b):
        pltpu.sync_copy(src_hbm.at[pl.ds(base + j * CH, CH)], sidx.at[b])
        pltpu.sync_copy(dst_hbm.at[pl.ds(base + j * CH, CH)], didx.at[b])
        pltpu.make_async_copy(z_hbm.at[sidx.at[b]], rows.at[b],
                              gsem[b]).start()

    for b in range(NBUF - 1):
        load_and_gather(b, b)

    def mb(g, carry):
        for b in range(NBUF):
            j = g * NBUF + b
            b3 = (b + NBUF - 1) % NBUF
            @pl.when((j + NBUF - 1 < NCH) & (j >= 1))
            def _drain():
                pltpu.make_async_copy(rows.at[b3], acc.at[didx.at[b3]],
                                      ssem[b3]).wait()

            @pl.when(j + NBUF - 1 < NCH)
            def _prefetch():
                load_and_gather(j + NBUF - 1, b3)

            pltpu.make_async_copy(z_hbm.at[sidx.at[b]], rows.at[b],
                                  gsem[b]).wait()
            pltpu.async_copy(rows.at[b], acc.at[didx.at[b]], ssem[b],
                             add=True)
        return carry

    lax.fori_loop(0, NCH // NBUF, mb, 0)
    for b in range(NBUF):
        pltpu.make_async_copy(rows.at[b], acc.at[didx.at[b]], ssem[b]).wait()
    plsc.subcore_barrier()
    pltpu.sync_copy(acc.at[pl.ds(s * RPT, RPT)],
                    out_hbm.at[c].at[pl.ds(s * RPT, RPT)])


@functools.cache
def _sc_kernels():
    mesh = plsc.VectorSubcoreMesh(core_axis_name="c", subcore_axis_name="s")
    cparams = pltpu.CompilerParams(needs_layout_passes=False)
    deg_kernel = pl.kernel(
        _deg_body,
        out_type=jax.ShapeDtypeStruct((NC, NS, NPAD), jnp.float32),
        mesh=mesh,
        scratch_types=[
            pltpu.VMEM((EP,), jnp.int32),
            pltpu.VMEM((NPAD,), jnp.float32),
        ],
        compiler_params=cparams,
    )
    msg_kernel = pl.kernel(
        _msg_body,
        out_type=jax.ShapeDtypeStruct((NC, NPAD, D), jnp.float32),
        mesh=mesh,
        scratch_types=(
            [pltpu.VMEM((NBUF, CH), jnp.int32),
             pltpu.VMEM((NBUF, CH), jnp.int32),
             pltpu.VMEM((NBUF, CH, D), jnp.float32),
             pltpu.VMEM_SHARED((NPAD, D), jnp.float32)]
            + [pltpu.SemaphoreType.DMA] * (2 * NBUF)),
        compiler_params=cparams,
    )
    return deg_kernel, msg_kernel


def _dinv_body(dp_ref, dinv_ref):
    tot = jnp.sum(dp_ref[...], axis=0, keepdims=True) + 1.0
    dinv_ref[...] = lax.rsqrt(tot)


def _dinv_call(dp):
    return pl.pallas_call(
        _dinv_body,
        out_shape=jax.ShapeDtypeStruct((1, NPAD), jnp.float32),
    )(dp)


_BM = 2000


def _xw_body(state_ref, w_ref, xw_ref):
    xw_ref[...] = jnp.dot(state_ref[...], w_ref[...],
                          preferred_element_type=jnp.float32,
                          precision=lax.Precision.HIGHEST)


def _xw_call(state, w_gcn):
    grid = (N // _BM,)
    return pl.pallas_call(
        _xw_body,
        grid=grid,
        in_specs=[
            pl.BlockSpec((_BM, D), lambda i: (i, 0)),
            pl.BlockSpec((D, D), lambda i: (0, 0)),
        ],
        out_specs=pl.BlockSpec((_BM, D), lambda i: (i, 0)),
        out_shape=jax.ShapeDtypeStruct((N, D), jnp.float32),
    )(state, w_gcn)


def _zscale_body(xw_ref, dinv_ref, z_ref):
    z_ref[...] = xw_ref[...] * dinv_ref[...]


def _zscale_call(xw, dinv_col):
    grid = (N // _BM,)
    return pl.pallas_call(
        _zscale_body,
        grid=grid,
        in_specs=[
            pl.BlockSpec((_BM, D), lambda i: (i, 0)),
            pl.BlockSpec((_BM, 1), lambda i: (i, 0)),
        ],
        out_specs=pl.BlockSpec((_BM, D), lambda i: (i, 0)),
        out_shape=jax.ShapeDtypeStruct((N, D), jnp.float32),
    )(xw, dinv_col)


_BMH = 2048
_GH = _BMH // ACT


def _head_body(acc0, acc1, z, state, dinv, aflat, bg, w1, b1, w2, b2, w3, b3,
               out):
    x = (acc0[...] + acc1[...] + z[...]) * dinv[...] + bg[...]
    x = jnp.maximum(x, 0.0) + state[...]
    gi = lax.broadcasted_iota(jnp.int32, (_GH, _BMH), 0)
    ri = lax.broadcasted_iota(jnp.int32, (_GH, _BMH), 1)
    sel = (ri // ACT) == gi
    smat = jnp.where(sel, aflat[...] * 10.0, 0.0)
    y = jnp.dot(smat, x, preferred_element_type=jnp.float32, precision=lax.Precision.HIGHEST)
    h = jnp.maximum(
        jnp.dot(y, w1[...], preferred_element_type=jnp.float32, precision=lax.Precision.HIGHEST) + b1[...], 0.0)
    h = jnp.maximum(
        jnp.dot(h, w2[...], preferred_element_type=jnp.float32, precision=lax.Precision.HIGHEST) + b2[...], 0.0)
    out[...] = jnp.dot(h, w3[...], preferred_element_type=jnp.float32, precision=lax.Precision.HIGHEST) + b3[...]


def _head_call(acc0, acc1, z, state, dinv_col, aflat, b_gcn, w1, b1, w2, b2,
               w3, b3):
    grid = (NPAD // _BMH,)
    row_spec = pl.BlockSpec((_BMH, D), lambda i: (i, 0))
    return pl.pallas_call(
        _head_body,
        grid=grid,
        in_specs=[
            row_spec, row_spec, row_spec, row_spec,
            pl.BlockSpec((_BMH, 1), lambda i: (i, 0)),
            pl.BlockSpec((1, _BMH), lambda i: (0, i)),
            pl.BlockSpec((1, D), lambda i: (0, 0)),
            pl.BlockSpec((D, H), lambda i: (0, 0)),
            pl.BlockSpec((1, H), lambda i: (0, 0)),
            pl.BlockSpec((H, H), lambda i: (0, 0)),
            pl.BlockSpec((1, H), lambda i: (0, 0)),
            pl.BlockSpec((H, 1), lambda i: (0, 0)),
            pl.BlockSpec((1, 1), lambda i: (0, 0)),
        ],
        out_specs=pl.BlockSpec((_GH, 1), lambda i: (i, 0)),
        out_shape=jax.ShapeDtypeStruct((NPAD // ACT, 1), jnp.float32),
    )(acc0, acc1, z, state, dinv_col, aflat, b_gcn, w1, b1, w2, b2, w3, b3)


def kernel(state, edge_index, action, W_gcn, b_gcn, W1, b1, W2, b2, W3, b3):
    src = edge_index[0]
    dst = edge_index[1]
    wpad = EP - E // NW
    src_p = jnp.concatenate(
        [src.reshape(NW, E // NW),
         jnp.zeros((NW, wpad), jnp.int32)], axis=1).reshape(-1)
    dummy_rows = jnp.broadcast_to(
        jnp.arange(N, N + wpad, dtype=jnp.int32), (NW, wpad))
    dst_p = jnp.concatenate(
        [dst.reshape(NW, E // NW), dummy_rows], axis=1).reshape(-1)

    deg_kernel, msg_kernel = _sc_kernels()
    deg_parts = deg_kernel(dst_p)
    xw = _xw_call(state, W_gcn)
    dinv_row = _dinv_call(deg_parts.reshape(NW, NPAD))
    dinv_full = dinv_row.reshape(NPAD, 1)
    dinv_col = dinv_full[:N]

    z = _zscale_call(xw, dinv_col)
    accp = msg_kernel(src_p, dst_p, z)

    rpad = NPAD - N
    zrows = jnp.zeros((rpad, D), jnp.float32)
    z_p = jnp.concatenate([z, zrows])
    state_p = jnp.concatenate([state, zrows])
    aflat = jnp.concatenate(
        [action.reshape(1, N), jnp.zeros((1, rpad), jnp.float32)], axis=1)

    out = _head_call(
        accp[0], accp[1], z_p, state_p, dinv_full, aflat,
        b_gcn.reshape(1, D),
        W1, b1.reshape(1, H), W2, b2.reshape(1, H), W3, b3.reshape(1, 1))
    return out.reshape(NPAD // ACT)[:N // ACT]


# speedup vs baseline: 1.4046x; 1.3136x over previous
import functools

import jax
import jax.numpy as jnp
from jax import lax
from jax.experimental import pallas as pl
from jax.experimental.pallas import tpu as pltpu
from jax.experimental.pallas import tpu_sc as plsc

N = 10000
D = 128
E = 320000
H = 256
ACT = 8

NC = 2
NS = 16
NW = NC * NS
CH = 128
EP = 10240
EPAD = NW * EP
NCH = EP // CH
NPAD = 10240
RPT = NPAD // NS
DUMMY = N + 100

def _deg_body(dst_hbm, out_hbm, dstbuf, locdeg):
    c = lax.axis_index("c")
    s = lax.axis_index("s")
    wid = s * NC + c
    zero16 = jnp.zeros((16,), jnp.float32)
    ones16 = jnp.ones((16,), jnp.float32)

    def zb(i, carry):
        locdeg[pl.ds(i * 16, 16)] = zero16
        return carry

    lax.fori_loop(0, NPAD // 16, zb, 0)
    pltpu.sync_copy(dst_hbm.at[pl.ds(wid * EP, EP)], dstbuf)

    def ab(i, carry):
        idx = dstbuf[pl.ds(i * 16, 16)]
        plsc.addupdate_scatter(locdeg, [idx], ones16)
        return carry

    lax.fori_loop(0, EP // 16, ab, 0)
    pltpu.sync_copy(locdeg, out_hbm.at[c].at[s])


NBUF = 2
HD = D // 2
EPT = EPAD // NS
NCHT = EPT // CH
NZR = NPAD // NS


def _msg_body(src_hbm, dst_hbm, z2_hbm, out_hbm, sidx, didx, rows, zsp, acc,
              gs0, gs1, ss0, ss1):
    c = lax.axis_index("c")
    s = lax.axis_index("s")
    zero16 = jnp.zeros((16,), jnp.float32)
    gsem = (gs0, gs1)
    ssem = (ss0, ss1)

    pltpu.sync_copy(z2_hbm.at[c].at[pl.ds(s * NZR, NZR)],
                    zsp.at[pl.ds(s * NZR, NZR)])

    def zb(i, carry):
        r = i // (HD // 16)
        k = i % (HD // 16)
        rows[0, r, pl.ds(k * 16, 16)] = zero16
        return carry

    lax.fori_loop(0, CH * HD // 16, zb, 0)
    for q in range(RPT // CH):
        pltpu.sync_copy(rows.at[0], acc.at[pl.ds(s * RPT + q * CH, CH)])
    plsc.subcore_barrier()

    base = s * EPT

    def load_and_gather(j, b):
        pltpu.sync_copy(src_hbm.at[pl.ds(base + j * CH, CH)], sidx.at[b])
        pltpu.sync_copy(dst_hbm.at[pl.ds(base + j * CH, CH)], didx.at[b])
        pltpu.make_async_copy(zsp.at[sidx.at[b]], rows.at[b],
                              gsem[b]).start()

    for b in range(NBUF - 1):
        load_and_gather(b, b)

    def mb(g, carry):
        for b in range(NBUF):
            j = g * NBUF + b
            b3 = (b + NBUF - 1) % NBUF
            @pl.when((j + NBUF - 1 < NCHT) & (j >= 1))
            def _drain():
                pltpu.make_async_copy(rows.at[b3], acc.at[didx.at[b3]],
                                      ssem[b3]).wait()

            @pl.when(j + NBUF - 1 < NCHT)
            def _prefetch():
                load_and_gather(j + NBUF - 1, b3)

            pltpu.make_async_copy(zsp.at[sidx.at[b]], rows.at[b],
                                  gsem[b]).wait()
            pltpu.async_copy(rows.at[b], acc.at[didx.at[b]], ssem[b],
                             add=True)
        return carry

    lax.fori_loop(0, NCHT // NBUF, mb, 0)
    for b in range(NBUF):
        pltpu.make_async_copy(rows.at[b], acc.at[didx.at[b]], ssem[b]).wait()
    plsc.subcore_barrier()
    pltpu.sync_copy(acc.at[pl.ds(s * RPT, RPT)],
                    out_hbm.at[c].at[pl.ds(s * RPT, RPT)])


@functools.cache
def _sc_kernels():
    mesh = plsc.VectorSubcoreMesh(core_axis_name="c", subcore_axis_name="s")
    cparams = pltpu.CompilerParams(needs_layout_passes=False)
    cparams_lin = pltpu.CompilerParams(needs_layout_passes=False,
                                       use_tc_tiling_on_sc=False)
    deg_kernel = pl.kernel(
        _deg_body,
        out_type=jax.ShapeDtypeStruct((NC, NS, NPAD), jnp.float32),
        mesh=mesh,
        scratch_types=[
            pltpu.VMEM((EP,), jnp.int32),
            pltpu.VMEM((NPAD,), jnp.float32),
        ],
        compiler_params=cparams,
    )
    msg_kernel = pl.kernel(
        _msg_body,
        out_type=jax.ShapeDtypeStruct((NC, NPAD, HD), jnp.float32),
        mesh=mesh,
        scratch_types=(
            [pltpu.VMEM((NBUF, CH), jnp.int32),
             pltpu.VMEM((NBUF, CH), jnp.int32),
             pltpu.VMEM((NBUF, CH, HD), jnp.float32),
             pltpu.VMEM_SHARED((NPAD, HD), jnp.float32),
             pltpu.VMEM_SHARED((NPAD, HD), jnp.float32)]
            + [pltpu.SemaphoreType.DMA] * (2 * NBUF)),
        compiler_params=cparams_lin,
    )
    return deg_kernel, msg_kernel


def _dinv_body(dp_ref, dinv_ref):
    tot = jnp.sum(dp_ref[...], axis=0, keepdims=True) + 1.0
    dinv_ref[...] = lax.rsqrt(tot)


def _dinv_call(dp):
    return pl.pallas_call(
        _dinv_body,
        out_shape=jax.ShapeDtypeStruct((1, NPAD), jnp.float32),
    )(dp)


_BM = 2000


def _xw_body(state_ref, w_ref, xw_ref):
    xw_ref[...] = jnp.dot(state_ref[...], w_ref[...],
                          preferred_element_type=jnp.float32,
                          precision=lax.Precision.HIGHEST)


def _xw_call(state, w_gcn):
    grid = (N // _BM,)
    return pl.pallas_call(
        _xw_body,
        grid=grid,
        in_specs=[
            pl.BlockSpec((_BM, D), lambda i: (i, 0)),
            pl.BlockSpec((D, D), lambda i: (0, 0)),
        ],
        out_specs=pl.BlockSpec((_BM, D), lambda i: (i, 0)),
        out_shape=jax.ShapeDtypeStruct((N, D), jnp.float32),
    )(state, w_gcn)


def _zscale_body(xw_ref, dinv_ref, z2_ref):
    z = xw_ref[...] * dinv_ref[...]
    z2_ref[0] = z[:, :HD]
    z2_ref[1] = z[:, HD:]


def _zscale_call(xw, dinv_col):
    grid = (N // _BM,)
    return pl.pallas_call(
        _zscale_body,
        grid=grid,
        in_specs=[
            pl.BlockSpec((_BM, D), lambda i: (i, 0)),
            pl.BlockSpec((_BM, 1), lambda i: (i, 0)),
        ],
        out_specs=pl.BlockSpec((NC, _BM, HD), lambda i: (0, i, 0)),
        out_shape=jax.ShapeDtypeStruct((NC, N, HD), jnp.float32),
    )(xw, dinv_col)


_BMH = 2048
_GH = _BMH // ACT


def _head_body(acc0, acc1, z0, z1, state, dinv, aflat, bg, w1, b1, w2, b2, w3,
               b3, out):
    x = jnp.concatenate([acc0[...] + z0[...], acc1[...] + z1[...]], axis=1)
    x = x * dinv[...] + bg[...]
    x = jnp.maximum(x, 0.0) + state[...]
    gi = lax.broadcasted_iota(jnp.int32, (_GH, _BMH), 0)
    ri = lax.broadcasted_iota(jnp.int32, (_GH, _BMH), 1)
    sel = (ri // ACT) == gi
    smat = jnp.where(sel, aflat[...] * 10.0, 0.0)
    y = jnp.dot(smat, x, preferred_element_type=jnp.float32, precision=lax.Precision.HIGHEST)
    h = jnp.maximum(
        jnp.dot(y, w1[...], preferred_element_type=jnp.float32, precision=lax.Precision.HIGHEST) + b1[...], 0.0)
    h = jnp.maximum(
        jnp.dot(h, w2[...], preferred_element_type=jnp.float32, precision=lax.Precision.HIGHEST) + b2[...], 0.0)
    out[...] = jnp.dot(h, w3[...], preferred_element_type=jnp.float32, precision=lax.Precision.HIGHEST) + b3[...]


def _head_call(acc0, acc1, z0, z1, state, dinv_col, aflat, b_gcn, w1, b1, w2,
               b2, w3, b3):
    grid = (NPAD // _BMH,)
    half_spec = pl.BlockSpec((_BMH, HD), lambda i: (i, 0))
    row_spec = pl.BlockSpec((_BMH, D), lambda i: (i, 0))
    return pl.pallas_call(
        _head_body,
        grid=grid,
        in_specs=[
            half_spec, half_spec, half_spec, half_spec, row_spec,
            pl.BlockSpec((_BMH, 1), lambda i: (i, 0)),
            pl.BlockSpec((1, _BMH), lambda i: (0, i)),
            pl.BlockSpec((1, D), lambda i: (0, 0)),
            pl.BlockSpec((D, H), lambda i: (0, 0)),
            pl.BlockSpec((1, H), lambda i: (0, 0)),
            pl.BlockSpec((H, H), lambda i: (0, 0)),
            pl.BlockSpec((1, H), lambda i: (0, 0)),
            pl.BlockSpec((H, 1), lambda i: (0, 0)),
            pl.BlockSpec((1, 1), lambda i: (0, 0)),
        ],
        out_specs=pl.BlockSpec((_GH, 1), lambda i: (i, 0)),
        out_shape=jax.ShapeDtypeStruct((NPAD // ACT, 1), jnp.float32),
    )(acc0, acc1, z0, z1, state, dinv_col, aflat, b_gcn, w1, b1, w2, b2, w3,
      b3)


def kernel(state, edge_index, action, W_gcn, b_gcn, W1, b1, W2, b2, W3, b3):
    src = edge_index[0]
    dst = edge_index[1]
    wpad = EP - E // NW
    src_p = jnp.concatenate(
        [src.reshape(NW, E // NW),
         jnp.zeros((NW, wpad), jnp.int32)], axis=1).reshape(-1)
    dummy_rows = jnp.broadcast_to(
        jnp.arange(N, N + wpad, dtype=jnp.int32), (NW, wpad))
    dst_p = jnp.concatenate(
        [dst.reshape(NW, E // NW), dummy_rows], axis=1).reshape(-1)

    deg_kernel, msg_kernel = _sc_kernels()
    deg_parts = deg_kernel(dst_p)
    xw = _xw_call(state, W_gcn)
    dinv_row = _dinv_call(deg_parts.reshape(NW, NPAD))
    dinv_full = dinv_row.reshape(NPAD, 1)
    dinv_col = dinv_full[:N]

    z2 = _zscale_call(xw, dinv_col)
    rpad = NPAD - N
    z2_p = jnp.concatenate(
        [z2, jnp.zeros((NC, rpad, HD), jnp.float32)], axis=1)
    accp = msg_kernel(src_p, dst_p, z2_p)

    z0_p = z2_p[0]
    z1_p = z2_p[1]
    state_p = jnp.concatenate([state, jnp.zeros((rpad, D), jnp.float32)])
    aflat = jnp.concatenate(
        [action.reshape(1, N), jnp.zeros((1, rpad), jnp.float32)], axis=1)

    out = _head_call(
        accp[0], accp[1], z0_p, z1_p, state_p, dinv_full, aflat,
        b_gcn.reshape(1, D),
        W1, b1.reshape(1, H), W2, b2.reshape(1, H), W3, b3.reshape(1, 1))
    return out.reshape(NPAD // ACT)[:N // ACT]


# speedup vs baseline: 1.4135x; 1.0063x over previous
import functools

import jax
import jax.numpy as jnp
from jax import lax
from jax.experimental import pallas as pl
from jax.experimental.pallas import tpu as pltpu
from jax.experimental.pallas import tpu_sc as plsc

N = 10000
D = 128
E = 320000
H = 256
ACT = 8

NC = 2
NS = 16
NW = NC * NS
CH = 128
EP = 10240
EPAD = NW * EP
NCH = EP // CH
NPAD = 10240
RPT = NPAD // NS
DUMMY = N + 100

def _deg_body(dst_hbm, out_hbm, dstbuf, locdeg):
    c = lax.axis_index("c")
    s = lax.axis_index("s")
    wid = s * NC + c
    zero16 = jnp.zeros((16,), jnp.float32)
    ones16 = jnp.ones((16,), jnp.float32)

    def zb(i, carry):
        locdeg[pl.ds(i * 16, 16)] = zero16
        return carry

    lax.fori_loop(0, NPAD // 16, zb, 0)
    pltpu.sync_copy(dst_hbm.at[pl.ds(wid * EP, EP)], dstbuf)

    def ab(i, carry):
        idx = dstbuf[pl.ds(i * 16, 16)]
        plsc.addupdate_scatter(locdeg, [idx], ones16)
        return carry

    lax.fori_loop(0, EP // 16, ab, 0)
    pltpu.sync_copy(locdeg, out_hbm.at[c].at[s])


NBUF = 4
HD = D // 2
EPT = EPAD // NS
NCHT = EPT // CH
NZR = NPAD // NS


def _msg_body(src_hbm, dst_hbm, z2_hbm, out_hbm, sidx, didx, rows, zsp, acc,
              gs0, gs1, gs2, gs3, ss0, ss1, ss2, ss3):
    c = lax.axis_index("c")
    s = lax.axis_index("s")
    zero16 = jnp.zeros((16,), jnp.float32)
    gsem = (gs0, gs1, gs2, gs3)
    ssem = (ss0, ss1, ss2, ss3)

    pltpu.sync_copy(z2_hbm.at[c].at[pl.ds(s * NZR, NZR)],
                    zsp.at[pl.ds(s * NZR, NZR)])

    def zb(i, carry):
        r = i // (HD // 16)
        k = i % (HD // 16)
        rows[0, r, pl.ds(k * 16, 16)] = zero16
        return carry

    lax.fori_loop(0, CH * HD // 16, zb, 0)
    for q in range(RPT // CH):
        pltpu.sync_copy(rows.at[0], acc.at[pl.ds(s * RPT + q * CH, CH)])
    plsc.subcore_barrier()

    base = s * EPT

    def load_and_gather(j, b):
        pltpu.sync_copy(src_hbm.at[pl.ds(base + j * CH, CH)], sidx.at[b])
        pltpu.sync_copy(dst_hbm.at[pl.ds(base + j * CH, CH)], didx.at[b])
        pltpu.make_async_copy(zsp.at[sidx.at[b]], rows.at[b],
                              gsem[b]).start()

    for b in range(NBUF - 1):
        load_and_gather(b, b)

    def mb(g, carry):
        for b in range(NBUF):
            j = g * NBUF + b
            b3 = (b + NBUF - 1) % NBUF
            @pl.when((j + NBUF - 1 < NCHT) & (j >= 1))
            def _drain():
                pltpu.make_async_copy(rows.at[b3], acc.at[didx.at[b3]],
                                      ssem[b3]).wait()

            @pl.when(j + NBUF - 1 < NCHT)
            def _prefetch():
                load_and_gather(j + NBUF - 1, b3)

            pltpu.make_async_copy(zsp.at[sidx.at[b]], rows.at[b],
                                  gsem[b]).wait()
            pltpu.async_copy(rows.at[b], acc.at[didx.at[b]], ssem[b],
                             add=True)
        return carry

    lax.fori_loop(0, NCHT // NBUF, mb, 0)
    for b in range(NBUF):
        pltpu.make_async_copy(rows.at[b], acc.at[didx.at[b]], ssem[b]).wait()
    plsc.subcore_barrier()
    pltpu.sync_copy(acc.at[pl.ds(s * RPT, RPT)],
                    out_hbm.at[c].at[pl.ds(s * RPT, RPT)])


@functools.cache
def _sc_kernels():
    mesh = plsc.VectorSubcoreMesh(core_axis_name="c", subcore_axis_name="s")
    cparams = pltpu.CompilerParams(needs_layout_passes=False)
    cparams_lin = pltpu.CompilerParams(needs_layout_passes=False,
                                       use_tc_tiling_on_sc=False)
    deg_kernel = pl.kernel(
        _deg_body,
        out_type=jax.ShapeDtypeStruct((NC, NS, NPAD), jnp.float32),
        mesh=mesh,
        scratch_types=[
            pltpu.VMEM((EP,), jnp.int32),
            pltpu.VMEM((NPAD,), jnp.float32),
        ],
        compiler_params=cparams,
    )
    msg_kernel = pl.kernel(
        _msg_body,
        out_type=jax.ShapeDtypeStruct((NC, NPAD, HD), jnp.float32),
        mesh=mesh,
        scratch_types=(
            [pltpu.VMEM((NBUF, CH), jnp.int32),
             pltpu.VMEM((NBUF, CH), jnp.int32),
             pltpu.VMEM((NBUF, CH, HD), jnp.float32),
             pltpu.VMEM_SHARED((NPAD, HD), jnp.float32),
             pltpu.VMEM_SHARED((NPAD, HD), jnp.float32)]
            + [pltpu.SemaphoreType.DMA] * (2 * NBUF)),
        compiler_params=cparams_lin,
    )
    return deg_kernel, msg_kernel


def _dinv_body(dp_ref, dinv_ref):
    tot = jnp.sum(dp_ref[...], axis=0, keepdims=True) + 1.0
    dinv_ref[...] = lax.rsqrt(tot)


def _dinv_call(dp):
    return pl.pallas_call(
        _dinv_body,
        out_shape=jax.ShapeDtypeStruct((1, NPAD), jnp.float32),
    )(dp)


_BM = 2048


def _xw_body(state_ref, w_ref, xw_ref):
    xw_ref[...] = jnp.dot(state_ref[...], w_ref[...],
                          preferred_element_type=jnp.float32,
                          precision=lax.Precision.HIGHEST)


def _xw_call(state, w_gcn):
    grid = (NPAD // _BM,)
    return pl.pallas_call(
        _xw_body,
        grid=grid,
        in_specs=[
            pl.BlockSpec((_BM, D), lambda i: (i, 0)),
            pl.BlockSpec((D, D), lambda i: (0, 0)),
        ],
        out_specs=pl.BlockSpec((_BM, D), lambda i: (i, 0)),
        out_shape=jax.ShapeDtypeStruct((NPAD, D), jnp.float32),
    )(state, w_gcn)


def _zscale_body(xw_ref, dinv_ref, z2_ref):
    z = xw_ref[...] * dinv_ref[...]
    z2_ref[0] = z[:, :HD]
    z2_ref[1] = z[:, HD:]


def _zscale_call(xw, dinv_col):
    grid = (NPAD // _BM,)
    return pl.pallas_call(
        _zscale_body,
        grid=grid,
        in_specs=[
            pl.BlockSpec((_BM, D), lambda i: (i, 0)),
            pl.BlockSpec((_BM, 1), lambda i: (i, 0)),
        ],
        out_specs=pl.BlockSpec((NC, _BM, HD), lambda i: (0, i, 0)),
        out_shape=jax.ShapeDtypeStruct((NC, NPAD, HD), jnp.float32),
    )(xw, dinv_col)


_BMH = 2048
_GH = _BMH // ACT


def _head_body(acc0, acc1, z0, z1, state, dinv, aflat, bg, w1, b1, w2, b2, w3,
               b3, out):
    x = jnp.concatenate([acc0[...] + z0[...], acc1[...] + z1[...]], axis=1)
    x = x * dinv[...] + bg[...]
    x = jnp.maximum(x, 0.0) + state[...]
    gi = lax.broadcasted_iota(jnp.int32, (_GH, _BMH), 0)
    ri = lax.broadcasted_iota(jnp.int32, (_GH, _BMH), 1)
    sel = (ri // ACT) == gi
    smat = jnp.where(sel, aflat[...] * 10.0, 0.0)
    y = jnp.dot(smat, x, preferred_element_type=jnp.float32, precision=lax.Precision.HIGHEST)
    h = jnp.maximum(
        jnp.dot(y, w1[...], preferred_element_type=jnp.float32, precision=lax.Precision.HIGHEST) + b1[...], 0.0)
    h = jnp.maximum(
        jnp.dot(h, w2[...], preferred_element_type=jnp.float32, precision=lax.Precision.HIGHEST) + b2[...], 0.0)
    out[...] = jnp.dot(h, w3[...], preferred_element_type=jnp.float32, precision=lax.Precision.HIGHEST) + b3[...]


def _head_call(acc0, acc1, z0, z1, state, dinv_col, aflat, b_gcn, w1, b1, w2,
               b2, w3, b3):
    grid = (NPAD // _BMH,)
    half_spec = pl.BlockSpec((_BMH, HD), lambda i: (i, 0))
    row_spec = pl.BlockSpec((_BMH, D), lambda i: (i, 0))
    return pl.pallas_call(
        _head_body,
        grid=grid,
        in_specs=[
            half_spec, half_spec, half_spec, half_spec, row_spec,
            pl.BlockSpec((_BMH, 1), lambda i: (i, 0)),
            pl.BlockSpec((1, _BMH), lambda i: (0, i)),
            pl.BlockSpec((1, D), lambda i: (0, 0)),
            pl.BlockSpec((D, H), lambda i: (0, 0)),
            pl.BlockSpec((1, H), lambda i: (0, 0)),
            pl.BlockSpec((H, H), lambda i: (0, 0)),
            pl.BlockSpec((1, H), lambda i: (0, 0)),
            pl.BlockSpec((H, 1), lambda i: (0, 0)),
            pl.BlockSpec((1, 1), lambda i: (0, 0)),
        ],
        out_specs=pl.BlockSpec((_GH, 1), lambda i: (i, 0)),
        out_shape=jax.ShapeDtypeStruct((NPAD // ACT, 1), jnp.float32),
    )(acc0, acc1, z0, z1, state, dinv_col, aflat, b_gcn, w1, b1, w2, b2, w3,
      b3)


def kernel(state, edge_index, action, W_gcn, b_gcn, W1, b1, W2, b2, W3, b3):
    src = edge_index[0]
    dst = edge_index[1]
    wpad = EP - E // NW
    src_p = jnp.concatenate(
        [src.reshape(NW, E // NW),
         jnp.zeros((NW, wpad), jnp.int32)], axis=1).reshape(-1)
    dummy_rows = jnp.broadcast_to(
        jnp.arange(N, N + wpad, dtype=jnp.int32), (NW, wpad))
    dst_p = jnp.concatenate(
        [dst.reshape(NW, E // NW), dummy_rows], axis=1).reshape(-1)

    deg_kernel, msg_kernel = _sc_kernels()
    rpad = NPAD - N
    state_p = jnp.concatenate([state, jnp.zeros((rpad, D), jnp.float32)])
    deg_parts = deg_kernel(dst_p)
    xw = _xw_call(state_p, W_gcn)
    dinv_row = _dinv_call(deg_parts.reshape(NW, NPAD))
    dinv_full = dinv_row.reshape(NPAD, 1)

    z2_p = _zscale_call(xw, dinv_full)
    accp = msg_kernel(src_p, dst_p, z2_p)

    z0_p = z2_p[0]
    z1_p = z2_p[1]
    aflat = jnp.concatenate(
        [action.reshape(1, N), jnp.zeros((1, rpad), jnp.float32)], axis=1)

    out = _head_call(
        accp[0], accp[1], z0_p, z1_p, state_p, dinv_full, aflat,
        b_gcn.reshape(1, D),
        W1, b1.reshape(1, H), W2, b2.reshape(1, H), W3, b3.reshape(1, 1))
    return out.reshape(NPAD // ACT)[:N // ACT]
